# baseline (speedup 1.0000x reference)
import functools

import jax
import jax.numpy as jnp
from jax import lax
from jax.experimental import pallas as pl
from jax.experimental.pallas import tpu as pltpu
from jax.experimental.pallas import tpu_sc as plsc

HMR = 3
BETA = 1.0


def _phase1_body(conf_ref, pbt_ref, dbt_ref, gt_ref, lab_ref, key_ref,
                 log_ref, sums_ref):
    nb = pl.program_id(1)

    conf = conf_ref[0]
    rmax = jnp.max(conf, axis=1)[None, :]

    lab = lab_ref[0]
    cc = jax.lax.dot_general(lab, conf, (((1,), (1,)), ((), ())),
                             preferred_element_type=jnp.float32)

    gt = gt_ref[0]
    gx0, gy0, gx1, gy1 = (gt[:, c:c + 1] for c in range(4))
    db = dbt_ref[nb]
    dx0, dy0, dx1, dy1 = (db[c:c + 1] for c in range(4))

    area_a = (dx1 - dx0) * (dy1 - dy0)
    area_g = (gx1 - gx0) * (gy1 - gy0)
    wx = jnp.clip(jnp.minimum(dx1, gx1) - jnp.maximum(dx0, gx0), 0.0, None)
    wy = jnp.clip(jnp.minimum(dy1, gy1) - jnp.maximum(dy0, gy0), 0.0, None)
    inter = wx * wy
    denom = area_a + area_g - inter + 1e-9
    matches = jnp.logical_and(inter >= 0.5 * denom, denom > 0.0)
    box_m = jnp.any(matches, axis=0, keepdims=True)

    pb = pbt_ref[0, nb]
    ssl1 = 0.0
    for c in range(4):
        d = pb[c:c + 1] - gt[:, c:c + 1]
        ad = jnp.abs(d)
        ssl1 = ssl1 + jnp.where(ad < BETA, 0.5 * d * d / BETA,
                                ad - 0.5 * BETA)
    ll = jnp.sum(jnp.where(matches, ssl1, 0.0))
    ml = jnp.sum(jnp.where(matches, jnp.log(cc), 0.0))

    negv = jnp.where(box_m, -1.0, rmax)
    key_ref[0, 0] = jax.lax.bitcast_convert_type(negv, jnp.int32)
    log_ref[0, 0] = jnp.where(box_m, 0.0, jnp.log(1.0 + rmax))

    lane = jax.lax.broadcasted_iota(jnp.int32, (1, 128), 1)
    contrib = (jnp.where(lane == 1, ll, 0.0)
               + jnp.where(lane == 2, ml, 0.0))

    @pl.when(nb == 0)
    def _():
        sums_ref[0] = contrib

    @pl.when(nb != 0)
    def _():
        sums_ref[0] = sums_ref[0] + contrib


def _sc_phase2_body(n, npad, keys_hbm, logs_hbm, out_hbm, keys_v, logs_v,
                    hist_v, row_v):
    wid = lax.axis_index("s") * 2 + lax.axis_index("c")
    nv = npad // 16

    @pl.when(wid < keys_hbm.shape[0])
    def _():
        pltpu.sync_copy(keys_hbm.at[wid], keys_v)
        pltpu.sync_copy(logs_hbm.at[wid], logs_v)
        ones = jnp.ones((16,), jnp.float32)

        def cnt_body(i, acc):
            k = keys_v[pl.ds(i * 16, 16)]
            return acc + jnp.sum(jnp.where(k < 0, 1.0, 0.0))

        tm = lax.fori_loop(0, nv, cnt_body, 0.0) - float(npad - n)
        div = jnp.where(tm > 0.0, tm, 1000.0)
        kk = jnp.minimum(HMR * div, float(n) - tm)

        def radix_pass(shift, state):
            lo, rem = state
            hi = lo + (1024 << shift)
            for i2 in range(1024 // 16):
                hist_v[pl.ds(i2 * 16, 16)] = jnp.zeros((16,), jnp.float32)

            def hbody(i, carry):
                k = keys_v[pl.ds(i * 16, 16)]
                ok = jnp.logical_and(k >= lo, k < hi)
                b_ = jnp.bitwise_and(
                    jax.lax.shift_right_logical(k, shift), 1023)
                plsc.addupdate_scatter(hist_v, [b_], ones, mask=ok)
                return carry

            lax.fori_loop(0, nv, hbody, 0)

            def sbody(i, c):
                cnt_above, jstar, sexcl_sel = c
                v = 63 - i
                h = hist_v[pl.ds(v * 16, 16)]
                sincl = lax.rev(plsc.cumsum(lax.rev(h, (0,))), (0,))
                sincl = sincl + cnt_above
                sexcl = sincl - h
                m = jnp.logical_and(sexcl < rem, sincl >= rem)
                ids = (lax.iota(jnp.int32, 16) + v * 16).astype(jnp.float32)
                jstar = jstar + jnp.sum(jnp.where(m, ids, 0.0))
                sexcl_sel = sexcl_sel + jnp.sum(jnp.where(m, sexcl, 0.0))
                cnt_above = cnt_above + jnp.sum(h)
                return (cnt_above, jstar, sexcl_sel)

            _, jstar_f, sexcl_sel = lax.fori_loop(0, 64, sbody,
                                                  (0.0, 0.0, 0.0))
            jstar = jstar_f.astype(jnp.int32)
            lo2 = lo + jax.lax.shift_left(jstar, shift)
            return (lo2, rem - sexcl_sel)

        state = (jnp.int32(0), kk)
        state = radix_pass(20, state)
        state = radix_pass(10, state)
        t, _ = radix_pass(0, state)

        def fbody(i, c):
            cgt, sgt, ceq, seq = c
            k = keys_v[pl.ds(i * 16, 16)]
            lg = logs_v[pl.ds(i * 16, 16)]
            mgt = k > t
            meq = k == t
            cgt = cgt + jnp.sum(jnp.where(mgt, 1.0, 0.0))
            sgt = sgt + jnp.sum(jnp.where(mgt, lg, 0.0))
            ceq = ceq + jnp.sum(jnp.where(meq, 1.0, 0.0))
            seq = seq + jnp.sum(jnp.where(meq, lg, 0.0))
            return (cgt, sgt, ceq, seq)

        cgt, sgt, ceq, seq = lax.fori_loop(0, nv, fbody,
                                           (0.0, 0.0, 0.0, 0.0))
        lane = lax.iota(jnp.int32, 16)
        row_v[...] = (jnp.where(lane == 0, sgt, 0.0)
                      + jnp.where(lane == 1, tm, 0.0)
                      + jnp.where(lane == 2, cgt, 0.0)
                      + jnp.where(lane == 3, ceq, 0.0)
                      + jnp.where(lane == 4, seq, 0.0)
                      + jnp.where(lane == 5, kk, 0.0))
        pltpu.sync_copy(row_v, out_hbm.at[wid])


def kernel(pred_boxes, pred_confidences, gt_boxes, gt_labels, default_boxes):
    b, n, c = pred_confidences.shape
    g = gt_boxes.shape[1]
    bn = 4000
    nblk = n // bn

    pb_t = (jnp.transpose(pred_boxes, (0, 2, 1))
            .reshape(b, 4, nblk, bn).transpose(0, 2, 1, 3))
    db_t = (jnp.transpose(default_boxes, (1, 0))
            .reshape(4, nblk, bn).transpose(1, 0, 2))
    lab_f = gt_labels.astype(jnp.float32)

    keys, logs, sums = pl.pallas_call(
        _phase1_body,
        grid=(b, nblk),
        in_specs=[
            pl.BlockSpec((1, bn, c), lambda i, j: (i, j, 0)),
            pl.BlockSpec((1, nblk, 4, bn), lambda i, j: (i, 0, 0, 0)),
            pl.BlockSpec((nblk, 4, bn), lambda i, j: (0, 0, 0)),
            pl.BlockSpec((1, g, 4), lambda i, j: (i, 0, 0)),
            pl.BlockSpec((1, g, c), lambda i, j: (i, 0, 0)),
        ],
        out_specs=[
            pl.BlockSpec((1, 1, 1, bn), lambda i, j: (i, j, 0, 0)),
            pl.BlockSpec((1, 1, 1, bn), lambda i, j: (i, j, 0, 0)),
            pl.BlockSpec((1, 1, 128), lambda i, j: (i, 0, 0)),
        ],
        out_shape=[
            jax.ShapeDtypeStruct((b, nblk, 1, bn), jnp.int32),
            jax.ShapeDtypeStruct((b, nblk, 1, bn), jnp.float32),
            jax.ShapeDtypeStruct((b, 1, 128), jnp.float32),
        ],
    )(pred_confidences, pb_t, db_t, gt_boxes, lab_f)

    npad = ((n + 127) // 128) * 128
    keys2 = jnp.pad(keys.reshape(b, n), ((0, 0), (0, npad - n)),
                    constant_values=-1)
    logs2 = jnp.pad(logs.reshape(b, n), ((0, 0), (0, npad - n)),
                    constant_values=0.0)

    mesh = plsc.VectorSubcoreMesh(core_axis_name="c", subcore_axis_name="s")
    sc2 = functools.partial(
        pl.kernel,
        out_type=jax.ShapeDtypeStruct((b, 16), jnp.float32),
        mesh=mesh,
        compiler_params=pltpu.CompilerParams(needs_layout_passes=False),
        scratch_types=[
            pltpu.VMEM((npad,), jnp.int32),
            pltpu.VMEM((npad,), jnp.float32),
            pltpu.VMEM((1024,), jnp.float32),
            pltpu.VMEM((16,), jnp.float32),
        ],
    )(functools.partial(_sc_phase2_body, n, npad))
    scout = sc2(keys2, logs2)

    sgt, tm, cgt, ceq, seq, kk = (scout[:, i] for i in range(6))
    logt = jnp.where(ceq > 0.0, seq / jnp.maximum(ceq, 1.0), 0.0)
    nm = jnp.where(kk > 0.0, sgt + (kk - cgt) * logt, 0.0)
    ll = sums[:, 0, 1]
    ml = sums[:, 0, 2]
    div = jnp.where(tm > 0.0, tm, 1000.0)
    return jnp.sum((-ml + nm + ll) / div)


# speedup vs baseline: 1.0232x; 1.0232x over previous
import functools

import jax
import jax.numpy as jnp
from jax import lax
from jax.experimental import pallas as pl
from jax.experimental.pallas import tpu as pltpu
from jax.experimental.pallas import tpu_sc as plsc

HMR = 3
BETA = 1.0


def _phase1_body(conf_ref, pbt_ref, dbt_ref, gt_ref, lab_ref, key_ref,
                 log_ref, sums_ref):
    nb = pl.program_id(1)

    conf = conf_ref[0]
    rmax = jnp.max(conf, axis=1)[None, :]

    lab = lab_ref[0]
    cc = jax.lax.dot_general(lab, conf, (((1,), (1,)), ((), ())),
                             preferred_element_type=jnp.float32)

    gt = gt_ref[0]
    gx0, gy0, gx1, gy1 = (gt[:, c:c + 1] for c in range(4))
    db = dbt_ref[nb]
    dx0, dy0, dx1, dy1 = (db[c:c + 1] for c in range(4))

    area_a = (dx1 - dx0) * (dy1 - dy0)
    area_g = (gx1 - gx0) * (gy1 - gy0)
    wx = jnp.clip(jnp.minimum(dx1, gx1) - jnp.maximum(dx0, gx0), 0.0, None)
    wy = jnp.clip(jnp.minimum(dy1, gy1) - jnp.maximum(dy0, gy0), 0.0, None)
    inter = wx * wy
    denom = area_a + area_g - inter + 1e-9
    matches = jnp.logical_and(inter >= 0.5 * denom, denom > 0.0)
    box_m = jnp.any(matches, axis=0, keepdims=True)

    pb = pbt_ref[0, nb]
    ssl1 = 0.0
    for c in range(4):
        d = pb[c:c + 1] - gt[:, c:c + 1]
        ad = jnp.abs(d)
        ssl1 = ssl1 + jnp.where(ad < BETA, 0.5 * d * d / BETA,
                                ad - 0.5 * BETA)
    ll = jnp.sum(jnp.where(matches, ssl1, 0.0))
    ml = jnp.sum(jnp.where(matches, jnp.log(cc), 0.0))

    negv = jnp.where(box_m, -1.0, rmax)
    key_ref[0, 0] = jax.lax.bitcast_convert_type(negv, jnp.int32)
    log_ref[0, 0] = jnp.where(box_m, 0.0, jnp.log(1.0 + rmax))

    lane = jax.lax.broadcasted_iota(jnp.int32, (1, 128), 1)
    contrib = (jnp.where(lane == 1, ll, 0.0)
               + jnp.where(lane == 2, ml, 0.0))

    @pl.when(nb == 0)
    def _():
        sums_ref[0] = contrib

    @pl.when(nb != 0)
    def _():
        sums_ref[0] = sums_ref[0] + contrib


def _sc_phase2_body(n, npad, keys_hbm, logs_hbm, out_hbm, keys_v, logs_v,
                    hist_v, row_v):
    wid = lax.axis_index("s") * 2 + lax.axis_index("c")
    nv = npad // 16

    @pl.when(wid < keys_hbm.shape[0])
    def _():
        pltpu.sync_copy(keys_hbm.at[wid], keys_v)
        pltpu.sync_copy(logs_hbm.at[wid], logs_v)
        ones = jnp.ones((16,), jnp.float32)
        unroll = 8

        def hist_fill(shift, lo, hi, count_negs):
            for i2 in range(1024 // 16):
                hist_v[pl.ds(i2 * 16, 16)] = jnp.zeros((16,), jnp.float32)

            def hbody(ii, acc):
                for u in range(unroll):
                    k = keys_v[pl.ds((ii * unroll + u) * 16, 16)]
                    ok = jnp.logical_and(k >= lo, k < hi)
                    b_ = jnp.bitwise_and(
                        jax.lax.shift_right_logical(k, shift), 1023)
                    plsc.addupdate_scatter(hist_v, [b_], ones, mask=ok)
                    if count_negs:
                        acc = acc + jnp.sum(jnp.where(k < 0, 1.0, 0.0))
                return acc

            return lax.fori_loop(0, nv // unroll, hbody, 0.0)

        def select(shift, lo, rem):
            def sbody(i, c):
                cnt_above, jstar, sexcl_sel = c
                v = 63 - i
                h = hist_v[pl.ds(v * 16, 16)]
                sincl = lax.rev(plsc.cumsum(lax.rev(h, (0,))), (0,))
                sincl = sincl + cnt_above
                sexcl = sincl - h
                m = jnp.logical_and(sexcl < rem, sincl >= rem)
                ids = (lax.iota(jnp.int32, 16) + v * 16).astype(jnp.float32)
                jstar = jstar + jnp.sum(jnp.where(m, ids, 0.0))
                sexcl_sel = sexcl_sel + jnp.sum(jnp.where(m, sexcl, 0.0))
                cnt_above = cnt_above + jnp.sum(h)
                return (cnt_above, jstar, sexcl_sel)

            _, jstar_f, sexcl_sel = lax.fori_loop(0, 64, sbody,
                                                  (0.0, 0.0, 0.0))
            jstar = jstar_f.astype(jnp.int32)
            return lo + jax.lax.shift_left(jstar, shift), rem - sexcl_sel

        lo = jnp.int32(0)
        negs = hist_fill(20, lo, jnp.int32(1 << 30), True)
        tm = negs - float(npad - n)
        div = jnp.where(tm > 0.0, tm, 1000.0)
        kk = jnp.minimum(HMR * div, float(n) - tm)

        lo, rem = select(20, lo, kk)
        hist_fill(10, lo, lo + (1024 << 10), False)
        lo, rem = select(10, lo, rem)
        hist_fill(0, lo, lo + 1024, False)
        t, _ = select(0, lo, rem)

        def fbody(ii, c):
            cgt, sgt, ceq, seq = c
            for u in range(unroll):
                k = keys_v[pl.ds((ii * unroll + u) * 16, 16)]
                lg = logs_v[pl.ds((ii * unroll + u) * 16, 16)]
                mgt = k > t
                meq = k == t
                cgt = cgt + jnp.sum(jnp.where(mgt, 1.0, 0.0))
                sgt = sgt + jnp.sum(jnp.where(mgt, lg, 0.0))
                ceq = ceq + jnp.sum(jnp.where(meq, 1.0, 0.0))
                seq = seq + jnp.sum(jnp.where(meq, lg, 0.0))
            return (cgt, sgt, ceq, seq)

        cgt, sgt, ceq, seq = lax.fori_loop(0, nv // unroll, fbody,
                                           (0.0, 0.0, 0.0, 0.0))
        lane = lax.iota(jnp.int32, 16)
        row_v[...] = (jnp.where(lane == 0, sgt, 0.0)
                      + jnp.where(lane == 1, tm, 0.0)
                      + jnp.where(lane == 2, cgt, 0.0)
                      + jnp.where(lane == 3, ceq, 0.0)
                      + jnp.where(lane == 4, seq, 0.0)
                      + jnp.where(lane == 5, kk, 0.0))
        pltpu.sync_copy(row_v, out_hbm.at[wid])


def kernel(pred_boxes, pred_confidences, gt_boxes, gt_labels, default_boxes):
    b, n, c = pred_confidences.shape
    g = gt_boxes.shape[1]
    bn = 4000
    nblk = n // bn

    pb_t = (jnp.transpose(pred_boxes, (0, 2, 1))
            .reshape(b, 4, nblk, bn).transpose(0, 2, 1, 3))
    db_t = (jnp.transpose(default_boxes, (1, 0))
            .reshape(4, nblk, bn).transpose(1, 0, 2))
    lab_f = gt_labels.astype(jnp.float32)

    keys, logs, sums = pl.pallas_call(
        _phase1_body,
        grid=(b, nblk),
        in_specs=[
            pl.BlockSpec((1, bn, c), lambda i, j: (i, j, 0)),
            pl.BlockSpec((1, nblk, 4, bn), lambda i, j: (i, 0, 0, 0)),
            pl.BlockSpec((nblk, 4, bn), lambda i, j: (0, 0, 0)),
            pl.BlockSpec((1, g, 4), lambda i, j: (i, 0, 0)),
            pl.BlockSpec((1, g, c), lambda i, j: (i, 0, 0)),
        ],
        out_specs=[
            pl.BlockSpec((1, 1, 1, bn), lambda i, j: (i, j, 0, 0)),
            pl.BlockSpec((1, 1, 1, bn), lambda i, j: (i, j, 0, 0)),
            pl.BlockSpec((1, 1, 128), lambda i, j: (i, 0, 0)),
        ],
        out_shape=[
            jax.ShapeDtypeStruct((b, nblk, 1, bn), jnp.int32),
            jax.ShapeDtypeStruct((b, nblk, 1, bn), jnp.float32),
            jax.ShapeDtypeStruct((b, 1, 128), jnp.float32),
        ],
    )(pred_confidences, pb_t, db_t, gt_boxes, lab_f)

    npad = ((n + 127) // 128) * 128
    keys2 = jnp.pad(keys.reshape(b, n), ((0, 0), (0, npad - n)),
                    constant_values=-1)
    logs2 = jnp.pad(logs.reshape(b, n), ((0, 0), (0, npad - n)),
                    constant_values=0.0)

    mesh = plsc.VectorSubcoreMesh(core_axis_name="c", subcore_axis_name="s")
    sc2 = functools.partial(
        pl.kernel,
        out_type=jax.ShapeDtypeStruct((b, 16), jnp.float32),
        mesh=mesh,
        compiler_params=pltpu.CompilerParams(needs_layout_passes=False),
        scratch_types=[
            pltpu.VMEM((npad,), jnp.int32),
            pltpu.VMEM((npad,), jnp.float32),
            pltpu.VMEM((1024,), jnp.float32),
            pltpu.VMEM((16,), jnp.float32),
        ],
    )(functools.partial(_sc_phase2_body, n, npad))
    scout = sc2(keys2, logs2)

    sgt, tm, cgt, ceq, seq, kk = (scout[:, i] for i in range(6))
    logt = jnp.where(ceq > 0.0, seq / jnp.maximum(ceq, 1.0), 0.0)
    nm = jnp.where(kk > 0.0, sgt + (kk - cgt) * logt, 0.0)
    ll = sums[:, 0, 1]
    ml = sums[:, 0, 2]
    div = jnp.where(tm > 0.0, tm, 1000.0)
    return jnp.sum((-ml + nm + ll) / div)
